# baseline (speedup 1.0000x reference)
import functools

import jax
import jax.numpy as jnp
from jax import lax
from jax.experimental import pallas as pl
from jax.experimental.pallas import tpu as pltpu
from jax.experimental.pallas import tpu_sc as plsc

B = 2
N = 512
NV = 20000
HID = 128
GRID = 32
CELLS = GRID * GRID * GRID
TOTAL = B * CELLS
NANCH = 512
TILE = 4096
N_SIDE = 8


def _r16(v):
    b = lax.bitcast_convert_type(v, jnp.int32)
    lsb = (b >> 13) & 1
    r = (b + 0x0FFF + lsb) & ~0x1FFF
    return lax.bitcast_convert_type(r, jnp.float32)


_NC = 2
_NS = 16
_NW = _NC * _NS
_NVP = 20480
_VPW = _NVP // _NW
_NIDX = _VPW // 128
_SLACK = 1024
_PTOT = TOTAL + _SLACK
_ZCH = _PTOT // _NS
_OCH = TOTAL // _NS


def _sc_mask_body(coords_hbm, cnt_hbm, coords_v, idx_v, ones_v, zero_v,
                  counts_sh):
    core = lax.axis_index("c")
    sid = lax.axis_index("s")
    gw = sid * _NC + core

    def _zero(j, _):
        zero_v[pl.ds(j * 16, 16)] = jnp.zeros((16,), jnp.float32)
        return 0

    lax.fori_loop(0, _ZCH // 16, _zero, 0)
    pltpu.sync_copy(zero_v, counts_sh.at[pl.ds(sid * _ZCH, _ZCH)])

    def _ones(j, _):
        ones_v[pl.ds(j * 16, 16)] = jnp.ones((16,), jnp.float32)
        return 0

    lax.fori_loop(0, 8, _ones, 0)

    pltpu.sync_copy(coords_hbm.at[:, pl.ds(gw * _VPW, _VPW)], coords_v)
    for t in range(_VPW // 16):
        s = t * 16
        bb = coords_v[0, pl.ds(s, 16)]
        xx = coords_v[1, pl.ds(s, 16)]
        yy = coords_v[2, pl.ds(s, 16)]
        zz = coords_v[3, pl.ds(s, 16)]
        lin = ((bb * GRID + xx) * GRID + yy) * GRID + zz
        idx_v[t // 8, pl.ds((t % 8) * 16, 16)] = lin

    plsc.subcore_barrier()

    for r in range(_NIDX):
        pltpu.sync_copy(ones_v, counts_sh.at[idx_v.at[r]], add=True)

    plsc.subcore_barrier()

    pltpu.sync_copy(counts_sh.at[pl.ds(sid * _OCH, _OCH)],
                    cnt_hbm.at[core, pl.ds(sid * _OCH, _OCH)])


def _sc_mask(coords_t):
    mesh = plsc.VectorSubcoreMesh(core_axis_name="c", subcore_axis_name="s")
    k = pl.kernel(
        _sc_mask_body,
        mesh=mesh,
        out_type=jax.ShapeDtypeStruct((_NC, TOTAL), jnp.float32),
        scratch_types=[
            pltpu.VMEM((4, _VPW), jnp.int32),
            pltpu.VMEM((_NIDX, 128), jnp.int32),
            pltpu.VMEM((128,), jnp.float32),
            pltpu.VMEM((_ZCH,), jnp.float32),
            pltpu.VMEM_SHARED((_PTOT,), jnp.float32),
        ],
        compiler_params=pltpu.CompilerParams(needs_layout_passes=False),
    )
    return k(coords_t)


_NXY = TILE // GRID
_NKEEP = 216
_NA = 224
_AXYALL = GRID * GRID


def _tc_prologue(xin_ref, w1_ref, b1_ref, w2_ref, b2_ref,
                 xb_ref, sxy_ref, szf_ref, exy_ref):
    xin = xin_ref[...]
    w1 = w1_ref[...]
    b1 = b1_ref[...]
    w2 = w2_ref[...]
    b2 = b2_ref[...]
    for b in range(B):
        h = jnp.maximum(
            jnp.dot(xin[b], w1, preferred_element_type=jnp.float32) + b1,
            0.0)
        xb = jnp.dot(h, w2, preferred_element_type=jnp.float32) + b2
        xsel = xb.reshape(N_SIDE, N_SIDE, N_SIDE, HID)[2:, 2:, 2:, :]
        xb_ref[b, :_NKEEP] = xsel.reshape(_NKEEP, HID).astype(jnp.bfloat16)
        xb_ref[b, _NKEEP:] = jnp.zeros((_NA - _NKEEP, HID), jnp.bfloat16)

    n = lax.broadcasted_iota(jnp.int32, (_NA, 1), 0)
    step = 100.0 / (N_SIDE - 1)
    pad = n >= _NKEEP
    ai = jnp.where(pad, 0, 2 + n // 36)
    aj = jnp.where(pad, 0, 2 + (n // 6) % 6)
    ak = jnp.where(pad, 0, 2 + n % 6)
    ax = jnp.where(pad, -100.0, _r16(-50.0 + ai.astype(jnp.float32) * step))
    ay = jnp.where(pad, -100.0, _r16(-50.0 + aj.astype(jnp.float32) * step))
    az = jnp.where(pad, -100.0, _r16(-50.0 + ak.astype(jnp.float32) * step))

    j = lax.broadcasted_iota(jnp.int32, (1, _AXYALL), 1)
    cxf = (j // GRID).astype(jnp.float32)
    cyf = (j % GRID).astype(jnp.float32)
    dx = _r16(cxf - ax)
    dy = _r16(cyf - ay)
    sxy_all = _r16(_r16(dx * dx) + _r16(dy * dy))
    for t in range(CELLS // TILE):
        sxy_ref[t] = sxy_all[:, t * _NXY:(t + 1) * _NXY]

    k = lax.broadcasted_iota(jnp.int32, (1, GRID), 1)
    dz = _r16(k.astype(jnp.float32) - az)
    sz = _r16(dz * dz)

    c = lax.broadcasted_iota(jnp.int32, (1, TILE), 1)
    exy_ref[...] = (
        c // GRID == lax.broadcasted_iota(jnp.int32, (_NXY, 1), 0)
    ).astype(jnp.float32)
    ez = (c % GRID == lax.broadcasted_iota(jnp.int32, (GRID, 1), 0)
          ).astype(jnp.float32)
    szf_ref[...] = jnp.dot(sz, ez, preferred_element_type=jnp.float32)


def _tc_body(xin_ref, w1_ref, b1_ref, w2_ref, b2_ref, cnt_ref, out_ref,
             xb_ref, sxy_ref, szf_ref, exy_ref):
    i = pl.program_id(0)

    @pl.when(i == 0)
    def _prologue():
        _tc_prologue(xin_ref, w1_ref, b1_ref, w2_ref, b2_ref,
                     xb_ref, sxy_ref, szf_ref, exy_ref)

    sxy = sxy_ref[i]
    sxy_f = jnp.dot(sxy, exy_ref[...], preferred_element_type=jnp.float32)
    ssum = sxy_f + szf_ref[...]
    dist = ssum * lax.rsqrt(ssum)
    e = jnp.exp(-dist)
    s = jnp.sum(e, axis=0, keepdims=True)
    rs = 1.0 / s
    eb = e.astype(jnp.bfloat16)

    for b in range(B):
        acc = lax.dot_general(
            xb_ref[b], eb, (((0,), (0,)), ((), ())),
            preferred_element_type=jnp.float32)
        occ = cnt_ref[b] + cnt_ref[B + b]
        masked = acc * jnp.where(occ > 0.0, rs, 0.0)
        out_ref[b] = masked.reshape(HID, TILE // _AXYALL, GRID, GRID)


def _tc_dense(xin, w1, b1, w2, b2, cnt4):
    grid = (CELLS // TILE,)
    xpt = TILE // _AXYALL
    return pl.pallas_call(
        _tc_body,
        grid=grid,
        in_specs=[
            pl.BlockSpec((B, N, 11), lambda i: (0, 0, 0)),
            pl.BlockSpec((11, HID), lambda i: (0, 0)),
            pl.BlockSpec((1, HID), lambda i: (0, 0)),
            pl.BlockSpec((HID, HID), lambda i: (0, 0)),
            pl.BlockSpec((1, HID), lambda i: (0, 0)),
            pl.BlockSpec((_NC * B, TILE), lambda i: (0, i)),
        ],
        out_specs=pl.BlockSpec(
            (B, HID, xpt, GRID, GRID), lambda i: (0, 0, i, 0, 0)),
        out_shape=jax.ShapeDtypeStruct((B, HID, GRID, GRID, GRID),
                                       jnp.float32),
        scratch_shapes=[
            pltpu.VMEM((B, _NA, HID), jnp.bfloat16),
            pltpu.VMEM((CELLS // TILE, _NA, _NXY), jnp.float32),
            pltpu.VMEM((_NA, TILE), jnp.float32),
            pltpu.VMEM((_NXY, TILE), jnp.float32),
        ],
    )(xin, w1, b1, w2, b2, cnt4)


def kernel(position, scale, rotation, opacity, voxel_coords, W1, b1, W2, b2):
    xin = jnp.concatenate([position, scale, rotation, opacity], axis=-1)
    pad = jnp.broadcast_to(
        jnp.array([[B, 0, 0, 0]], jnp.int32), (_NVP - NV, 4))
    coords_t = jnp.concatenate([voxel_coords, pad], axis=0).T
    cnt = _sc_mask(coords_t)
    cnt4 = cnt.reshape(_NC * B, CELLS)
    return _tc_dense(xin, W1, b1.reshape(1, HID), W2, b2.reshape(1, HID),
                     cnt4)


# speedup vs baseline: 3.5536x; 3.5536x over previous
import functools

import jax
import jax.numpy as jnp
from jax import lax
from jax.experimental import pallas as pl
from jax.experimental.pallas import tpu as pltpu
from jax.experimental.pallas import tpu_sc as plsc

B = 2
N = 512
NV = 20000
HID = 128
GRID = 32
CELLS = GRID * GRID * GRID
TOTAL = B * CELLS
NANCH = 512
TILE = 4096
N_SIDE = 8


def _r16(v):
    b = lax.bitcast_convert_type(v, jnp.int32)
    lsb = (b >> 13) & 1
    r = (b + 0x0FFF + lsb) & ~0x1FFF
    return lax.bitcast_convert_type(r, jnp.float32)


_NC = 2
_NS = 16
_NW = _NC * _NS
_NVP = 20480
_VPW = _NVP // _NW
_NIDX = _VPW // 128
_SLACK = 1024
_PTOT = TOTAL + _SLACK
_ZCH = _PTOT // _NS
_OCH = TOTAL // _NS


def _sc_mask_body(coords_hbm, cnt_hbm, coords_v, idx_v, ones_v, zero_v,
                  counts_sh):
    core = lax.axis_index("c")
    sid = lax.axis_index("s")
    gw = sid * _NC + core

    def _zero(j, _):
        zero_v[pl.ds(j * 16, 16)] = jnp.zeros((16,), jnp.float32)
        return 0

    lax.fori_loop(0, _ZCH // 16, _zero, 0)
    pltpu.sync_copy(zero_v, counts_sh.at[pl.ds(sid * _ZCH, _ZCH)])

    def _ones(j, _):
        ones_v[pl.ds(j * 16, 16)] = jnp.ones((16,), jnp.float32)
        return 0

    lax.fori_loop(0, 8, _ones, 0)

    pltpu.sync_copy(coords_hbm.at[:, pl.ds(gw * _VPW, _VPW)], coords_v)
    for t in range(_VPW // 16):
        s = t * 16
        bb = coords_v[0, pl.ds(s, 16)]
        xx = coords_v[1, pl.ds(s, 16)]
        yy = coords_v[2, pl.ds(s, 16)]
        zz = coords_v[3, pl.ds(s, 16)]
        lin = ((bb * GRID + xx) * GRID + yy) * GRID + zz
        idx_v[t // 8, pl.ds((t % 8) * 16, 16)] = lin

    plsc.subcore_barrier()

    for r in range(_NIDX):
        pltpu.sync_copy(ones_v, counts_sh.at[idx_v.at[r]], add=True)

    plsc.subcore_barrier()

    pltpu.sync_copy(counts_sh.at[pl.ds(sid * _OCH, _OCH)],
                    cnt_hbm.at[core, pl.ds(sid * _OCH, _OCH)])


def _sc_mask(coords_t):
    mesh = plsc.VectorSubcoreMesh(core_axis_name="c", subcore_axis_name="s")
    k = pl.kernel(
        _sc_mask_body,
        mesh=mesh,
        out_type=jax.ShapeDtypeStruct((_NC, TOTAL), jnp.float32),
        scratch_types=[
            pltpu.VMEM((4, _VPW), jnp.int32),
            pltpu.VMEM((_NIDX, 128), jnp.int32),
            pltpu.VMEM((128,), jnp.float32),
            pltpu.VMEM((_ZCH,), jnp.float32),
            pltpu.VMEM_SHARED((_PTOT,), jnp.float32),
        ],
        compiler_params=pltpu.CompilerParams(needs_layout_passes=False),
    )
    return k(coords_t)


_NXY = TILE // GRID
_NKEEP = 216
_NA = 224
_AXYALL = GRID * GRID


def _tc_prologue(xin_ref, w1_ref, b1_ref, w2_ref, b2_ref,
                 xb_ref, sxy_ref, szf_ref, exy_ref):
    xin = xin_ref[...]
    w1 = w1_ref[...]
    b1 = b1_ref[...]
    w2 = w2_ref[...]
    b2 = b2_ref[...]
    for b in range(B):
        h = jnp.maximum(
            jnp.dot(xin[b], w1, preferred_element_type=jnp.float32) + b1,
            0.0)
        xb = jnp.dot(h, w2, preferred_element_type=jnp.float32) + b2
        xsel = xb.reshape(N_SIDE, N_SIDE, N_SIDE, HID)[2:, 2:, 2:, :]
        xb_ref[b, :_NKEEP] = xsel.reshape(_NKEEP, HID).astype(jnp.bfloat16)
        xb_ref[b, _NKEEP:] = jnp.zeros((_NA - _NKEEP, HID), jnp.bfloat16)

    n = lax.broadcasted_iota(jnp.int32, (_NA, 1), 0)
    step = 100.0 / (N_SIDE - 1)
    pad = n >= _NKEEP
    ai = jnp.where(pad, 0, 2 + n // 36)
    aj = jnp.where(pad, 0, 2 + (n // 6) % 6)
    ak = jnp.where(pad, 0, 2 + n % 6)
    ax = jnp.where(pad, -100.0, _r16(-50.0 + ai.astype(jnp.float32) * step))
    ay = jnp.where(pad, -100.0, _r16(-50.0 + aj.astype(jnp.float32) * step))
    az = jnp.where(pad, -100.0, _r16(-50.0 + ak.astype(jnp.float32) * step))

    j = lax.broadcasted_iota(jnp.int32, (1, _AXYALL), 1)
    cxf = (j // GRID).astype(jnp.float32)
    cyf = (j % GRID).astype(jnp.float32)
    dx = _r16(cxf - ax)
    dy = _r16(cyf - ay)
    sxy_all = _r16(_r16(dx * dx) + _r16(dy * dy))
    for t in range(CELLS // TILE):
        sxy_ref[t] = sxy_all[:, t * _NXY:(t + 1) * _NXY]

    k = lax.broadcasted_iota(jnp.int32, (1, GRID), 1)
    dz = _r16(k.astype(jnp.float32) - az)
    sz = _r16(dz * dz)

    c = lax.broadcasted_iota(jnp.int32, (1, TILE), 1)
    exy_ref[...] = (
        c // GRID == lax.broadcasted_iota(jnp.int32, (_NXY, 1), 0)
    ).astype(jnp.float32)
    ez = (c % GRID == lax.broadcasted_iota(jnp.int32, (GRID, 1), 0)
          ).astype(jnp.float32)
    szf_ref[...] = jnp.dot(sz, ez, preferred_element_type=jnp.float32)


def _tc_body(xin_ref, w1_ref, b1_ref, w2_ref, b2_ref, cnt_ref, out_ref,
             xb_ref, sxy_ref, szf_ref, exy_ref):
    i = pl.program_id(0)

    @pl.when(i == 0)
    def _prologue():
        _tc_prologue(xin_ref, w1_ref, b1_ref, w2_ref, b2_ref,
                     xb_ref, sxy_ref, szf_ref, exy_ref)

    sxy = sxy_ref[i]
    sxy_f = jnp.dot(sxy, exy_ref[...], preferred_element_type=jnp.float32)
    ssum = sxy_f + szf_ref[...]
    dist = ssum * lax.rsqrt(ssum)
    e = jnp.exp(-dist)
    s = jnp.sum(e, axis=0, keepdims=True)
    rs = 1.0 / s

    for b in range(B):
        occ = cnt_ref[b] + cnt_ref[B + b]
        eb = (e * jnp.where(occ > 0.0, rs, 0.0)).astype(jnp.bfloat16)
        out_ref[b] = lax.dot_general(
            eb, xb_ref[b], (((0,), (0,)), ((), ())),
            preferred_element_type=jnp.float32)


def _tc_dense(xin, w1, b1, w2, b2, cnt4):
    grid = (CELLS // TILE,)
    return pl.pallas_call(
        _tc_body,
        grid=grid,
        in_specs=[
            pl.BlockSpec((B, N, 11), lambda i: (0, 0, 0)),
            pl.BlockSpec((11, HID), lambda i: (0, 0)),
            pl.BlockSpec((1, HID), lambda i: (0, 0)),
            pl.BlockSpec((HID, HID), lambda i: (0, 0)),
            pl.BlockSpec((1, HID), lambda i: (0, 0)),
            pl.BlockSpec((_NC * B, TILE), lambda i: (0, i)),
        ],
        out_specs=pl.BlockSpec((B, TILE, HID), lambda i: (0, i, 0)),
        out_shape=jax.ShapeDtypeStruct((B, CELLS, HID), jnp.float32),
        scratch_shapes=[
            pltpu.VMEM((B, _NA, HID), jnp.bfloat16),
            pltpu.VMEM((CELLS // TILE, _NA, _NXY), jnp.float32),
            pltpu.VMEM((_NA, TILE), jnp.float32),
            pltpu.VMEM((_NXY, TILE), jnp.float32),
        ],
    )(xin, w1, b1, w2, b2, cnt4)


def kernel(position, scale, rotation, opacity, voxel_coords, W1, b1, W2, b2):
    xin = jnp.concatenate([position, scale, rotation, opacity], axis=-1)
    pad = jnp.broadcast_to(
        jnp.array([[B, 0, 0, 0]], jnp.int32), (_NVP - NV, 4))
    coords_t = jnp.concatenate([voxel_coords, pad], axis=0).T
    cnt = _sc_mask(coords_t)
    cnt4 = cnt.reshape(_NC * B, CELLS)
    out = _tc_dense(xin, W1, b1.reshape(1, HID), W2, b2.reshape(1, HID), cnt4)
    return out.transpose(0, 2, 1).reshape(B, HID, GRID, GRID, GRID)
